# Initial kernel scaffold; baseline (speedup 1.0000x reference)
import jax
import jax.numpy as jnp
from jax.experimental import pallas as pl


def kernel(x, gate_w, w1, w2):
    raise NotImplementedError("write your pallas kernel here")


# speedup vs baseline: 2.6514x; 2.6514x over previous
import functools

import jax
import jax.numpy as jnp
from jax import lax
from jax.experimental import pallas as pl
from jax.experimental.pallas import tpu as pltpu
from jax.experimental.pallas import tpu_sc as plsc

E = 8
TOP_K = 2
H = 1024
D_FF = 4096
T = 2048
N_ASSIGN = T * TOP_K
BLK = 256
NBLK = N_ASSIGN // BLK + E
NPAD = NBLK * BLK

_NC, _NS = 2, 16
_NW = _NC * _NS


def _router_body(x_ref, g_ref, pos_ref, rw_ref, be_ref, aux_ref):
    xf = x_ref[...]
    gw = g_ref[...]
    logits = lax.dot_general(xf, gw, (((1,), (1,)), ((), ())),
                             preferred_element_type=jnp.float32)
    m = jnp.max(logits, axis=1, keepdims=True)
    ex = jnp.exp(logits - m)
    probs = ex / jnp.sum(ex, axis=1, keepdims=True)

    iota_e = lax.broadcasted_iota(jnp.int32, (T, E), 1).astype(jnp.float32)
    m1 = jnp.max(probs, axis=1, keepdims=True)
    e1 = jnp.min(jnp.where(probs == m1, iota_e, 99.0), axis=1, keepdims=True)
    probs2 = jnp.where(iota_e == e1, -1.0, probs)
    m2 = jnp.max(probs2, axis=1, keepdims=True)
    e2 = jnp.min(jnp.where(probs2 == m2, iota_e, 99.0), axis=1, keepdims=True)
    ssum = m1 + m2
    rw_ref[...] = jnp.concatenate([m1 / ssum, m2 / ssum], axis=0)

    eall = jnp.concatenate([e1, e2], axis=0)
    iota_e2 = lax.broadcasted_iota(jnp.int32, (N_ASSIGN, E), 1).astype(jnp.float32)
    oh = (eall == iota_e2).astype(jnp.float32)

    c = oh
    sh = 1
    while sh < N_ASSIGN:
        c = c + jnp.concatenate(
            [jnp.zeros((sh, E), jnp.float32), c[:N_ASSIGN - sh]], axis=0)
        sh *= 2
    rank_incl = jnp.sum(c * oh, axis=1, keepdims=True)
    counts = c[N_ASSIGN - 1:N_ASSIGN, :]

    counts_i = counts.astype(jnp.int32)
    padded = (((counts_i + (BLK - 1)) >> 8) << 8).astype(jnp.float32)
    incl = padded
    for s2 in (1, 2, 4):
        incl = incl + jnp.concatenate(
            [jnp.zeros((1, s2), jnp.float32), incl[:, :E - s2]], axis=1)
    excl = incl - padded
    base = jnp.sum(oh * excl, axis=1, keepdims=True)
    pos_ref[...] = (base + rank_incl - 1.0).astype(jnp.int32)

    blk_start = lax.broadcasted_iota(jnp.int32, (NBLK, E), 0).astype(jnp.float32) * BLK
    becnt = jnp.sum((blk_start >= incl).astype(jnp.float32), axis=1,
                    keepdims=True)
    total = incl[0:1, E - 1:E]
    active = blk_start[:, 0:1] < total
    be_ref[...] = jnp.where(active, becnt, 15.0).astype(jnp.int32)

    pmean = jnp.sum(probs, axis=0, keepdims=True) * (1.0 / T)
    f_i = counts * (1.0 / T)
    aux_ref[...] = E * jnp.sum(f_i * pmean, axis=1, keepdims=True)


def _run_router(xf, gate_w):
    return pl.pallas_call(
        _router_body,
        out_shape=(
            jax.ShapeDtypeStruct((N_ASSIGN, 1), jnp.int32),
            jax.ShapeDtypeStruct((N_ASSIGN, 1), jnp.float32),
            jax.ShapeDtypeStruct((NBLK, 1), jnp.int32),
            jax.ShapeDtypeStruct((1, 1), jnp.float32),
        ),
    )(xf, gate_w)


_DCH = 32
_PER_W = N_ASSIGN // _NW

@functools.lru_cache(maxsize=None)
def _make_dispatch():
    mesh = plsc.VectorSubcoreMesh(
        core_axis_name="c", subcore_axis_name="s",
        num_cores=_NC, num_subcores=_NS)
    return pl.kernel(
        _dispatch_body,
        out_type=jax.ShapeDtypeStruct((NPAD, H), jnp.float32),
        mesh=mesh,
        scratch_types=[
            pltpu.VMEM((_DCH,), jnp.int32),
            pltpu.VMEM((_DCH, H), jnp.float32),
            pltpu.SemaphoreType.DMA,
        ],
    )


def _dispatch_body(xf_hbm, pos_hbm, xs_hbm, idx_v, rows_v, sem):
    wid = lax.axis_index("s") * _NC + lax.axis_index("c")
    base = wid * _PER_W
    for ch in range(_PER_W // _DCH):
        j0 = base + ch * _DCH
        pltpu.sync_copy(pos_hbm.at[pl.ds(j0, _DCH)], idx_v)
        tok0 = jnp.where(j0 >= T, j0 - T, j0)
        pltpu.sync_copy(xf_hbm.at[pl.ds(tok0, _DCH)], rows_v)
        pltpu.async_copy(rows_v, xs_hbm.at[idx_v], sem).wait()


def _gelu_exact(x):
    z = jnp.abs(x) * 0.7071067811865476
    t = 1.0 / (1.0 + 0.3275911 * z)
    poly = t * (0.254829592 + t * (-0.284496736 + t * (
        1.421413741 + t * (-1.453152027 + t * 1.061405429))))
    erf_abs = 1.0 - poly * jnp.exp(-z * z)
    erf = jnp.where(x < 0.0, -erf_abs, erf_abs)
    return 0.5 * x * (1.0 + erf)


def _ffn_body(be_s, xs_ref, w1_ref, w2_ref, ys_ref):
    i = pl.program_id(0)

    @pl.when(be_s[i] < E)
    def _():
        xb = xs_ref[...].astype(jnp.bfloat16)
        hdn = lax.dot_general(xb, w1_ref[...], (((1,), (1,)), ((), ())),
                              preferred_element_type=jnp.float32)
        hdn = _gelu_exact(hdn).astype(jnp.bfloat16)
        ys_ref[...] = lax.dot_general(hdn, w2_ref[...],
                                      (((1,), (1,)), ((), ())),
                                      preferred_element_type=jnp.float32)


def _run_ffn(be, xs, w1r, w2r):
    grid_spec = pltpu.PrefetchScalarGridSpec(
        num_scalar_prefetch=1,
        grid=(NBLK,),
        in_specs=[
            pl.BlockSpec((BLK, H), lambda i, be: (i, 0)),
            pl.BlockSpec((D_FF, H), lambda i, be: (jnp.minimum(be[i], E - 1), 0)),
            pl.BlockSpec((H, D_FF), lambda i, be: (jnp.minimum(be[i], E - 1), 0)),
        ],
        out_specs=pl.BlockSpec((BLK, H), lambda i, be: (i, 0)),
    )
    return pl.pallas_call(
        _ffn_body,
        grid_spec=grid_spec,
        out_shape=jax.ShapeDtypeStruct((NPAD, H), jnp.float32),
        compiler_params=pltpu.CompilerParams(
            dimension_semantics=("arbitrary",)),
    )(be, xs, w1r, w2r)


_CCH = 16
_TPW = T // _NW


@functools.lru_cache(maxsize=None)
def _make_combine():
    mesh = plsc.VectorSubcoreMesh(
        core_axis_name="c", subcore_axis_name="s",
        num_cores=_NC, num_subcores=_NS)
    return pl.kernel(
        _combine_body,
        out_type=jax.ShapeDtypeStruct((T, H), jnp.float32),
        mesh=mesh,
        scratch_types=[
            pltpu.VMEM((_CCH,), jnp.int32),
            pltpu.VMEM((_CCH,), jnp.int32),
            pltpu.VMEM((_CCH,), jnp.float32),
            pltpu.VMEM((_CCH,), jnp.float32),
            pltpu.VMEM((_CCH, H), jnp.float32),
            pltpu.VMEM((_CCH, H), jnp.float32),
            pltpu.SemaphoreType.DMA,
            pltpu.SemaphoreType.DMA,
        ],
    )


def _combine_body(pos_hbm, rw_hbm, ys_hbm, out_hbm,
                  i0_v, i1_v, w0_v, w1_v, a_v, b_v, s0, s1):
    wid = lax.axis_index("s") * _NC + lax.axis_index("c")
    for ch in range(_TPW // _CCH):
        tb = wid * _TPW + ch * _CCH
        pltpu.sync_copy(pos_hbm.at[pl.ds(tb, _CCH)], i0_v)
        pltpu.sync_copy(pos_hbm.at[pl.ds(T + tb, _CCH)], i1_v)
        pltpu.sync_copy(rw_hbm.at[pl.ds(tb, _CCH)], w0_v)
        pltpu.sync_copy(rw_hbm.at[pl.ds(T + tb, _CCH)], w1_v)
        cp0 = pltpu.async_copy(ys_hbm.at[i0_v], a_v, s0)
        cp1 = pltpu.async_copy(ys_hbm.at[i1_v], b_v, s1)
        cp0.wait()
        cp1.wait()
        wa = w0_v[...]
        wb = w1_v[...]
        for r in range(_CCH):
            ridx = jnp.full((16,), r, jnp.int32)
            sa = wa.at[ridx].get(mode="promise_in_bounds")
            sb = wb.at[ridx].get(mode="promise_in_bounds")

            def body(cc, _, r=r, sa=sa, sb=sb):
                col = cc * 64
                for u in range(4):
                    av = a_v[r, pl.ds(col + u * 16, 16)]
                    bv = b_v[r, pl.ds(col + u * 16, 16)]
                    a_v[r, pl.ds(col + u * 16, 16)] = sa * av + sb * bv
                return 0

            lax.fori_loop(0, H // 64, body, 0)
        pltpu.sync_copy(a_v, out_hbm.at[pl.ds(tb, _CCH)])


def kernel(x, gate_w, w1, w2):
    b, s, h = x.shape
    xf = x.reshape(T, H)
    pos, rw, be, aux = _run_router(xf, gate_w)
    pos1 = pos.reshape(N_ASSIGN)
    rw1 = rw.reshape(N_ASSIGN)
    be1 = be.reshape(NBLK)
    xs = _make_dispatch()(xf, pos1)
    ys = _run_ffn(be1, xs,
                  w1.reshape(E * D_FF, H).astype(jnp.bfloat16),
                  w2.reshape(E * H, D_FF).astype(jnp.bfloat16))
    out = _make_combine()(pos1, rw1, ys)
    return out.reshape(b, s, h), aux.reshape(())


# speedup vs baseline: 2.9816x; 1.1245x over previous
import functools

import jax
import jax.numpy as jnp
from jax import lax
from jax.experimental import pallas as pl
from jax.experimental.pallas import tpu as pltpu
from jax.experimental.pallas import tpu_sc as plsc

E = 8
TOP_K = 2
H = 1024
D_FF = 4096
T = 2048
N_ASSIGN = T * TOP_K
BLK = 256
NBLK = N_ASSIGN // BLK + E
NPAD = NBLK * BLK

_NC, _NS = 2, 16
_NW = _NC * _NS


def _router_body(x_ref, g_ref, pos_ref, rw_ref, be_ref, aux_ref):
    xf = x_ref[...]
    gw = g_ref[...]
    logits = lax.dot_general(xf, gw, (((1,), (1,)), ((), ())),
                             preferred_element_type=jnp.float32)
    m = jnp.max(logits, axis=1, keepdims=True)
    ex = jnp.exp(logits - m)
    probs = ex / jnp.sum(ex, axis=1, keepdims=True)

    iota_e = lax.broadcasted_iota(jnp.int32, (T, E), 1).astype(jnp.float32)
    m1 = jnp.max(probs, axis=1, keepdims=True)
    e1 = jnp.min(jnp.where(probs == m1, iota_e, 99.0), axis=1, keepdims=True)
    probs2 = jnp.where(iota_e == e1, -1.0, probs)
    m2 = jnp.max(probs2, axis=1, keepdims=True)
    e2 = jnp.min(jnp.where(probs2 == m2, iota_e, 99.0), axis=1, keepdims=True)
    ssum = m1 + m2
    rw_ref[...] = jnp.concatenate([m1 / ssum, m2 / ssum], axis=0)

    eall = jnp.concatenate([e1, e2], axis=0)
    iota_e2 = lax.broadcasted_iota(jnp.int32, (N_ASSIGN, E), 1).astype(jnp.float32)
    oh = (eall == iota_e2).astype(jnp.float32)

    c = oh
    sh = 1
    while sh < N_ASSIGN:
        c = c + jnp.concatenate(
            [jnp.zeros((sh, E), jnp.float32), c[:N_ASSIGN - sh]], axis=0)
        sh *= 2
    rank_incl = jnp.sum(c * oh, axis=1, keepdims=True)
    counts = c[N_ASSIGN - 1:N_ASSIGN, :]

    counts_i = counts.astype(jnp.int32)
    padded = (((counts_i + (BLK - 1)) >> 8) << 8).astype(jnp.float32)
    incl = padded
    for s2 in (1, 2, 4):
        incl = incl + jnp.concatenate(
            [jnp.zeros((1, s2), jnp.float32), incl[:, :E - s2]], axis=1)
    excl = incl - padded
    base = jnp.sum(oh * excl, axis=1, keepdims=True)
    pos_ref[...] = (base + rank_incl - 1.0).astype(jnp.int32)

    blk_start = lax.broadcasted_iota(jnp.int32, (NBLK, E), 0).astype(jnp.float32) * BLK
    becnt = jnp.sum((blk_start >= incl).astype(jnp.float32), axis=1,
                    keepdims=True)
    total = incl[0:1, E - 1:E]
    active = blk_start[:, 0:1] < total
    be_ref[...] = jnp.where(active, becnt, 15.0).astype(jnp.int32)

    pmean = jnp.sum(probs, axis=0, keepdims=True) * (1.0 / T)
    f_i = counts * (1.0 / T)
    aux_ref[...] = E * jnp.sum(f_i * pmean, axis=1, keepdims=True)


def _run_router(xf, gate_w):
    return pl.pallas_call(
        _router_body,
        out_shape=(
            jax.ShapeDtypeStruct((N_ASSIGN, 1), jnp.int32),
            jax.ShapeDtypeStruct((N_ASSIGN, 1), jnp.float32),
            jax.ShapeDtypeStruct((NBLK, 1), jnp.int32),
            jax.ShapeDtypeStruct((1, 1), jnp.float32),
        ),
    )(xf, gate_w)


_DCH = 32
_PER_W = N_ASSIGN // _NW

@functools.lru_cache(maxsize=None)
def _make_dispatch():
    mesh = plsc.VectorSubcoreMesh(
        core_axis_name="c", subcore_axis_name="s",
        num_cores=_NC, num_subcores=_NS)
    return pl.kernel(
        _dispatch_body,
        out_type=jax.ShapeDtypeStruct((NPAD, H), jnp.float32),
        mesh=mesh,
        scratch_types=[
            pltpu.VMEM((_DCH,), jnp.int32),
            pltpu.VMEM((_DCH, H), jnp.float32),
            pltpu.SemaphoreType.DMA,
        ],
    )


def _dispatch_body(xf_hbm, pos_hbm, xs_hbm, idx_v, rows_v, sem):
    wid = lax.axis_index("s") * _NC + lax.axis_index("c")
    base = wid * _PER_W
    for ch in range(_PER_W // _DCH):
        j0 = base + ch * _DCH
        pltpu.sync_copy(pos_hbm.at[pl.ds(j0, _DCH)], idx_v)
        tok0 = jnp.where(j0 >= T, j0 - T, j0)
        pltpu.sync_copy(xf_hbm.at[pl.ds(tok0, _DCH)], rows_v)
        pltpu.async_copy(rows_v, xs_hbm.at[idx_v], sem).wait()


def _gelu_exact(x):
    return 0.5 * x * (1.0 + lax.erf(x * 0.7071067811865476))


def _ffn_body(be_s, xs_ref, w1_ref, w2_ref, ys_ref):
    i = pl.program_id(0)

    @pl.when(be_s[i] < E)
    def _():
        xb = xs_ref[...].astype(jnp.bfloat16)
        hdn = lax.dot_general(xb, w1_ref[...], (((1,), (1,)), ((), ())),
                              preferred_element_type=jnp.float32)
        hdn = _gelu_exact(hdn).astype(jnp.bfloat16)
        ys_ref[...] = lax.dot_general(hdn, w2_ref[...],
                                      (((1,), (1,)), ((), ())),
                                      preferred_element_type=jnp.float32)


def _run_ffn(be, xs, w1r, w2r):
    grid_spec = pltpu.PrefetchScalarGridSpec(
        num_scalar_prefetch=1,
        grid=(NBLK,),
        in_specs=[
            pl.BlockSpec((BLK, H), lambda i, be: (i, 0)),
            pl.BlockSpec((D_FF, H), lambda i, be: (jnp.minimum(be[i], E - 1), 0)),
            pl.BlockSpec((H, D_FF), lambda i, be: (jnp.minimum(be[i], E - 1), 0)),
        ],
        out_specs=pl.BlockSpec((BLK, H), lambda i, be: (i, 0)),
    )
    return pl.pallas_call(
        _ffn_body,
        grid_spec=grid_spec,
        out_shape=jax.ShapeDtypeStruct((NPAD, H), jnp.float32),
        compiler_params=pltpu.CompilerParams(
            dimension_semantics=("arbitrary",)),
    )(be, xs, w1r, w2r)


_CCH = 16
_TPW = T // _NW


@functools.lru_cache(maxsize=None)
def _make_combine():
    mesh = plsc.VectorSubcoreMesh(
        core_axis_name="c", subcore_axis_name="s",
        num_cores=_NC, num_subcores=_NS)
    return pl.kernel(
        _combine_body,
        out_type=jax.ShapeDtypeStruct((T, H), jnp.float32),
        mesh=mesh,
        scratch_types=[
            pltpu.VMEM((_CCH,), jnp.int32),
            pltpu.VMEM((_CCH,), jnp.int32),
            pltpu.VMEM((_CCH,), jnp.float32),
            pltpu.VMEM((_CCH,), jnp.float32),
            pltpu.VMEM((_CCH, H), jnp.float32),
            pltpu.VMEM((_CCH, H), jnp.float32),
            pltpu.SemaphoreType.DMA,
            pltpu.SemaphoreType.DMA,
        ],
    )


def _combine_body(pos_hbm, rw_hbm, ys_hbm, out_hbm,
                  i0_v, i1_v, w0_v, w1_v, a_v, b_v, s0, s1):
    wid = lax.axis_index("s") * _NC + lax.axis_index("c")
    for ch in range(_TPW // _CCH):
        tb = wid * _TPW + ch * _CCH
        pltpu.sync_copy(pos_hbm.at[pl.ds(tb, _CCH)], i0_v)
        pltpu.sync_copy(pos_hbm.at[pl.ds(T + tb, _CCH)], i1_v)
        pltpu.sync_copy(rw_hbm.at[pl.ds(tb, _CCH)], w0_v)
        pltpu.sync_copy(rw_hbm.at[pl.ds(T + tb, _CCH)], w1_v)
        cp0 = pltpu.async_copy(ys_hbm.at[i0_v], a_v, s0)
        cp1 = pltpu.async_copy(ys_hbm.at[i1_v], b_v, s1)
        cp0.wait()
        cp1.wait()
        wa = w0_v[...]
        wb = w1_v[...]
        for r in range(_CCH):
            ridx = jnp.full((16,), r, jnp.int32)
            sa = wa.at[ridx].get(mode="promise_in_bounds")
            sb = wb.at[ridx].get(mode="promise_in_bounds")

            def body(cc, _, r=r, sa=sa, sb=sb):
                col = cc * 64
                for u in range(4):
                    av = a_v[r, pl.ds(col + u * 16, 16)]
                    bv = b_v[r, pl.ds(col + u * 16, 16)]
                    a_v[r, pl.ds(col + u * 16, 16)] = sa * av + sb * bv
                return 0

            lax.fori_loop(0, H // 64, body, 0)
        pltpu.sync_copy(a_v, out_hbm.at[pl.ds(tb, _CCH)])


def kernel(x, gate_w, w1, w2):
    b, s, h = x.shape
    xf = x.reshape(T, H)
    pos, rw, be, aux = _run_router(xf, gate_w)
    pos1 = pos.reshape(N_ASSIGN)
    rw1 = rw.reshape(N_ASSIGN)
    be1 = be.reshape(NBLK)
    xs = _make_dispatch()(xf, pos1)
    ys = _run_ffn(be1, xs,
                  w1.reshape(E * D_FF, H).astype(jnp.bfloat16),
                  w2.reshape(E * H, D_FF).astype(jnp.bfloat16))
    out = _make_combine()(pos1, rw1, ys)
    return out.reshape(b, s, h), aux.reshape(())


# speedup vs baseline: 3.2782x; 1.0995x over previous
import functools

import jax
import jax.numpy as jnp
from jax import lax
from jax.experimental import pallas as pl
from jax.experimental.pallas import tpu as pltpu
from jax.experimental.pallas import tpu_sc as plsc

E = 8
TOP_K = 2
H = 1024
D_FF = 4096
T = 2048
N_ASSIGN = T * TOP_K
BLK = 256
NBLK = N_ASSIGN // BLK + E
NPAD = NBLK * BLK

_NC, _NS = 2, 16
_NW = _NC * _NS


def _router_body(x_ref, g_ref, pos_ref, rw_ref, be_ref, aux_ref):
    xf = x_ref[...]
    gw = g_ref[...]
    logits = lax.dot_general(xf, gw, (((1,), (1,)), ((), ())),
                             preferred_element_type=jnp.float32)
    m = jnp.max(logits, axis=1, keepdims=True)
    ex = jnp.exp(logits - m)
    probs = ex / jnp.sum(ex, axis=1, keepdims=True)

    iota_e = lax.broadcasted_iota(jnp.int32, (T, E), 1).astype(jnp.float32)
    m1 = jnp.max(probs, axis=1, keepdims=True)
    e1 = jnp.min(jnp.where(probs == m1, iota_e, 99.0), axis=1, keepdims=True)
    probs2 = jnp.where(iota_e == e1, -1.0, probs)
    m2 = jnp.max(probs2, axis=1, keepdims=True)
    e2 = jnp.min(jnp.where(probs2 == m2, iota_e, 99.0), axis=1, keepdims=True)
    ssum = m1 + m2
    rw_ref[...] = jnp.concatenate([m1 / ssum, m2 / ssum], axis=0)

    eall = jnp.concatenate([e1, e2], axis=0)
    iota_e2 = lax.broadcasted_iota(jnp.int32, (N_ASSIGN, E), 1).astype(jnp.float32)
    oh = (eall == iota_e2).astype(jnp.float32)

    c = oh
    sh = 1
    while sh < N_ASSIGN:
        c = c + jnp.concatenate(
            [jnp.zeros((sh, E), jnp.float32), c[:N_ASSIGN - sh]], axis=0)
        sh *= 2
    rank_incl = jnp.sum(c * oh, axis=1, keepdims=True)
    counts = c[N_ASSIGN - 1:N_ASSIGN, :]

    counts_i = counts.astype(jnp.int32)
    padded = (((counts_i + (BLK - 1)) >> 8) << 8).astype(jnp.float32)
    incl = padded
    for s2 in (1, 2, 4):
        incl = incl + jnp.concatenate(
            [jnp.zeros((1, s2), jnp.float32), incl[:, :E - s2]], axis=1)
    excl = incl - padded
    base = jnp.sum(oh * excl, axis=1, keepdims=True)
    pos_ref[...] = (base + rank_incl - 1.0).astype(jnp.int32)

    blk_start = lax.broadcasted_iota(jnp.int32, (NBLK, E), 0).astype(jnp.float32) * BLK
    becnt = jnp.sum((blk_start >= incl).astype(jnp.float32), axis=1,
                    keepdims=True)
    total = incl[0:1, E - 1:E]
    active = blk_start[:, 0:1] < total
    be_ref[...] = jnp.where(active, becnt, 15.0).astype(jnp.int32)

    pmean = jnp.sum(probs, axis=0, keepdims=True) * (1.0 / T)
    f_i = counts * (1.0 / T)
    aux_ref[...] = E * jnp.sum(f_i * pmean, axis=1, keepdims=True)


def _run_router(xf, gate_w):
    return pl.pallas_call(
        _router_body,
        out_shape=(
            jax.ShapeDtypeStruct((N_ASSIGN, 1), jnp.int32),
            jax.ShapeDtypeStruct((N_ASSIGN, 1), jnp.float32),
            jax.ShapeDtypeStruct((NBLK, 1), jnp.int32),
            jax.ShapeDtypeStruct((1, 1), jnp.float32),
        ),
    )(xf, gate_w)


_DCH = 32
_PER_W = N_ASSIGN // _NW

@functools.lru_cache(maxsize=None)
def _make_dispatch():
    mesh = plsc.VectorSubcoreMesh(
        core_axis_name="c", subcore_axis_name="s",
        num_cores=_NC, num_subcores=_NS)
    return pl.kernel(
        _dispatch_body,
        out_type=jax.ShapeDtypeStruct((NPAD, H), jnp.float32),
        mesh=mesh,
        scratch_types=[
            pltpu.VMEM((_DCH,), jnp.int32),
            pltpu.VMEM((_DCH, H), jnp.float32),
            pltpu.SemaphoreType.DMA,
        ],
    )


def _dispatch_body(xf_hbm, pos_hbm, xs_hbm, idx_v, rows_v, sem):
    wid = lax.axis_index("s") * _NC + lax.axis_index("c")
    base = wid * _PER_W
    for ch in range(_PER_W // _DCH):
        j0 = base + ch * _DCH
        pltpu.sync_copy(pos_hbm.at[pl.ds(j0, _DCH)], idx_v)
        tok0 = jnp.where(j0 >= T, j0 - T, j0)
        pltpu.sync_copy(xf_hbm.at[pl.ds(tok0, _DCH)], rows_v)
        pltpu.async_copy(rows_v, xs_hbm.at[idx_v], sem).wait()


def _gelu_exact(x):
    return 0.5 * x * (1.0 + lax.erf(x * 0.7071067811865476))


def _expert_changed(be_s, i):
    be = be_s[i]
    prev = be_s[jnp.maximum(i - 1, 0)]
    return jnp.logical_or(i == 0, be != prev)


def _ffn1_body(be_s, xs_ref, w1_ref, hdn_ref, w1c_ref):
    i = pl.program_id(0)
    active = be_s[i] < E

    @pl.when(jnp.logical_and(active, _expert_changed(be_s, i)))
    def _():
        w1c_ref[...] = w1_ref[...].astype(jnp.bfloat16)

    @pl.when(active)
    def _():
        xb = xs_ref[...].astype(jnp.bfloat16)
        hdn = lax.dot_general(xb, w1c_ref[...], (((1,), (1,)), ((), ())),
                              preferred_element_type=jnp.float32)
        hdn_ref[...] = _gelu_exact(hdn).astype(jnp.bfloat16)


def _ffn2_body(be_s, hdn_ref, w2_ref, ys_ref, w2c_ref):
    i = pl.program_id(0)
    active = be_s[i] < E

    @pl.when(jnp.logical_and(active, _expert_changed(be_s, i)))
    def _():
        w2c_ref[...] = w2_ref[...].astype(jnp.bfloat16)

    @pl.when(active)
    def _():
        ys_ref[...] = lax.dot_general(hdn_ref[...], w2c_ref[...],
                                      (((1,), (1,)), ((), ())),
                                      preferred_element_type=jnp.float32)


def _run_ffn(be, xs, w1r, w2r):
    hdn = pl.pallas_call(
        _ffn1_body,
        grid_spec=pltpu.PrefetchScalarGridSpec(
            num_scalar_prefetch=1,
            grid=(NBLK,),
            in_specs=[
                pl.BlockSpec((BLK, H), lambda i, be: (i, 0)),
                pl.BlockSpec((D_FF, H),
                             lambda i, be: (jnp.minimum(be[i], E - 1), 0)),
            ],
            out_specs=pl.BlockSpec((BLK, D_FF), lambda i, be: (i, 0)),
            scratch_shapes=[pltpu.VMEM((D_FF, H), jnp.bfloat16)],
        ),
        out_shape=jax.ShapeDtypeStruct((NPAD, D_FF), jnp.bfloat16),
        compiler_params=pltpu.CompilerParams(
            dimension_semantics=("arbitrary",)),
    )(be, xs, w1r)
    return pl.pallas_call(
        _ffn2_body,
        grid_spec=pltpu.PrefetchScalarGridSpec(
            num_scalar_prefetch=1,
            grid=(NBLK,),
            in_specs=[
                pl.BlockSpec((BLK, D_FF), lambda i, be: (i, 0)),
                pl.BlockSpec((H, D_FF),
                             lambda i, be: (jnp.minimum(be[i], E - 1), 0)),
            ],
            out_specs=pl.BlockSpec((BLK, H), lambda i, be: (i, 0)),
            scratch_shapes=[pltpu.VMEM((H, D_FF), jnp.bfloat16)],
        ),
        out_shape=jax.ShapeDtypeStruct((NPAD, H), jnp.float32),
        compiler_params=pltpu.CompilerParams(
            dimension_semantics=("arbitrary",)),
    )(be, hdn, w2r)


_CCH = 16
_TPW = T // _NW


@functools.lru_cache(maxsize=None)
def _make_combine():
    mesh = plsc.VectorSubcoreMesh(
        core_axis_name="c", subcore_axis_name="s",
        num_cores=_NC, num_subcores=_NS)
    return pl.kernel(
        _combine_body,
        out_type=jax.ShapeDtypeStruct((T, H), jnp.float32),
        mesh=mesh,
        scratch_types=[
            pltpu.VMEM((_CCH,), jnp.int32),
            pltpu.VMEM((_CCH,), jnp.int32),
            pltpu.VMEM((_CCH,), jnp.float32),
            pltpu.VMEM((_CCH,), jnp.float32),
            pltpu.VMEM((_CCH, H), jnp.float32),
            pltpu.VMEM((_CCH, H), jnp.float32),
            pltpu.SemaphoreType.DMA,
            pltpu.SemaphoreType.DMA,
        ],
    )


def _combine_body(pos_hbm, rw_hbm, ys_hbm, out_hbm,
                  i0_v, i1_v, w0_v, w1_v, a_v, b_v, s0, s1):
    wid = lax.axis_index("s") * _NC + lax.axis_index("c")
    for ch in range(_TPW // _CCH):
        tb = wid * _TPW + ch * _CCH
        pltpu.sync_copy(pos_hbm.at[pl.ds(tb, _CCH)], i0_v)
        pltpu.sync_copy(pos_hbm.at[pl.ds(T + tb, _CCH)], i1_v)
        pltpu.sync_copy(rw_hbm.at[pl.ds(tb, _CCH)], w0_v)
        pltpu.sync_copy(rw_hbm.at[pl.ds(T + tb, _CCH)], w1_v)
        cp0 = pltpu.async_copy(ys_hbm.at[i0_v], a_v, s0)
        cp1 = pltpu.async_copy(ys_hbm.at[i1_v], b_v, s1)
        cp0.wait()
        cp1.wait()
        wa = w0_v[...]
        wb = w1_v[...]
        for r in range(_CCH):
            ridx = jnp.full((16,), r, jnp.int32)
            sa = wa.at[ridx].get(mode="promise_in_bounds")
            sb = wb.at[ridx].get(mode="promise_in_bounds")

            def body(cc, _, r=r, sa=sa, sb=sb):
                col = cc * 64
                for u in range(4):
                    av = a_v[r, pl.ds(col + u * 16, 16)]
                    bv = b_v[r, pl.ds(col + u * 16, 16)]
                    a_v[r, pl.ds(col + u * 16, 16)] = sa * av + sb * bv
                return 0

            lax.fori_loop(0, H // 64, body, 0)
        pltpu.sync_copy(a_v, out_hbm.at[pl.ds(tb, _CCH)])


def kernel(x, gate_w, w1, w2):
    b, s, h = x.shape
    xf = x.reshape(T, H)
    pos, rw, be, aux = _run_router(xf, gate_w)
    pos1 = pos.reshape(N_ASSIGN)
    rw1 = rw.reshape(N_ASSIGN)
    be1 = be.reshape(NBLK)
    xs = _make_dispatch()(xf, pos1)
    ys = _run_ffn(be1, xs, w1.reshape(E * D_FF, H), w2.reshape(E * H, D_FF))
    out = _make_combine()(pos1, rw1, ys)
    return out.reshape(b, s, h), aux.reshape(())


# speedup vs baseline: 3.3719x; 1.0286x over previous
import functools

import jax
import jax.numpy as jnp
from jax import lax
from jax.experimental import pallas as pl
from jax.experimental.pallas import tpu as pltpu
from jax.experimental.pallas import tpu_sc as plsc

E = 8
TOP_K = 2
H = 1024
D_FF = 4096
T = 2048
N_ASSIGN = T * TOP_K
BLK = 256
NBLK = N_ASSIGN // BLK + E
NPAD = NBLK * BLK

_NC, _NS = 2, 16
_NW = _NC * _NS


def _router_body(x_ref, g_ref, pos_ref, rw_ref, be_ref, aux_ref):
    xf = x_ref[...]
    gw = g_ref[...]
    logits = lax.dot_general(xf, gw, (((1,), (1,)), ((), ())),
                             preferred_element_type=jnp.float32)
    m = jnp.max(logits, axis=1, keepdims=True)
    ex = jnp.exp(logits - m)
    probs = ex / jnp.sum(ex, axis=1, keepdims=True)

    iota_e = lax.broadcasted_iota(jnp.int32, (T, E), 1).astype(jnp.float32)
    m1 = jnp.max(probs, axis=1, keepdims=True)
    e1 = jnp.min(jnp.where(probs == m1, iota_e, 99.0), axis=1, keepdims=True)
    probs2 = jnp.where(iota_e == e1, -1.0, probs)
    m2 = jnp.max(probs2, axis=1, keepdims=True)
    e2 = jnp.min(jnp.where(probs2 == m2, iota_e, 99.0), axis=1, keepdims=True)
    ssum = m1 + m2
    rw_ref[...] = jnp.concatenate([m1 / ssum, m2 / ssum], axis=0)

    eall = jnp.concatenate([e1, e2], axis=0)
    iota_e2 = lax.broadcasted_iota(jnp.int32, (N_ASSIGN, E), 1).astype(jnp.float32)
    oh = (eall == iota_e2).astype(jnp.float32)

    c = oh
    sh = 1
    while sh < N_ASSIGN:
        c = c + jnp.concatenate(
            [jnp.zeros((sh, E), jnp.float32), c[:N_ASSIGN - sh]], axis=0)
        sh *= 2
    rank_incl = jnp.sum(c * oh, axis=1, keepdims=True)
    counts = c[N_ASSIGN - 1:N_ASSIGN, :]

    counts_i = counts.astype(jnp.int32)
    padded = (((counts_i + (BLK - 1)) >> 8) << 8).astype(jnp.float32)
    incl = padded
    for s2 in (1, 2, 4):
        incl = incl + jnp.concatenate(
            [jnp.zeros((1, s2), jnp.float32), incl[:, :E - s2]], axis=1)
    excl = incl - padded
    base = jnp.sum(oh * excl, axis=1, keepdims=True)
    pos_ref[...] = (base + rank_incl - 1.0).astype(jnp.int32)

    blk_start = lax.broadcasted_iota(jnp.int32, (NBLK, E), 0).astype(jnp.float32) * BLK
    becnt = jnp.sum((blk_start >= incl).astype(jnp.float32), axis=1,
                    keepdims=True)
    total = incl[0:1, E - 1:E]
    active = blk_start[:, 0:1] < total
    be_ref[...] = jnp.where(active, becnt, 15.0).astype(jnp.int32)

    pmean = jnp.sum(probs, axis=0, keepdims=True) * (1.0 / T)
    f_i = counts * (1.0 / T)
    aux_ref[...] = E * jnp.sum(f_i * pmean, axis=1, keepdims=True)


def _run_router(xf, gate_w):
    return pl.pallas_call(
        _router_body,
        out_shape=(
            jax.ShapeDtypeStruct((N_ASSIGN, 1), jnp.int32),
            jax.ShapeDtypeStruct((N_ASSIGN, 1), jnp.float32),
            jax.ShapeDtypeStruct((NBLK, 1), jnp.int32),
            jax.ShapeDtypeStruct((1, 1), jnp.float32),
        ),
    )(xf, gate_w)


_DCH = 32
_PER_W = N_ASSIGN // _NW

@functools.lru_cache(maxsize=None)
def _make_dispatch():
    mesh = plsc.VectorSubcoreMesh(
        core_axis_name="c", subcore_axis_name="s",
        num_cores=_NC, num_subcores=_NS)
    return pl.kernel(
        _dispatch_body,
        out_type=jax.ShapeDtypeStruct((NPAD, H), jnp.float32),
        mesh=mesh,
        scratch_types=[
            pltpu.VMEM((_DCH,), jnp.int32),
            pltpu.VMEM((_DCH,), jnp.int32),
            pltpu.VMEM((_DCH, H), jnp.float32),
            pltpu.VMEM((_DCH, H), jnp.float32),
            pltpu.SemaphoreType.DMA,
            pltpu.SemaphoreType.DMA,
            pltpu.SemaphoreType.DMA,
            pltpu.SemaphoreType.DMA,
        ],
    )


def _dispatch_body(xf_hbm, pos_hbm, xs_hbm,
                   idx0_v, idx1_v, rows0_v, rows1_v, ld0, ld1, st0, st1):
    wid = lax.axis_index("s") * _NC + lax.axis_index("c")
    base = wid * _PER_W
    nch = _PER_W // _DCH
    bufs = ((idx0_v, rows0_v, ld0, st0), (idx1_v, rows1_v, ld1, st1))
    scat = [None, None]
    for ch in range(nch):
        idx_v, rows_v, ld, st = bufs[ch % 2]
        if ch >= 2:
            scat[ch % 2].wait()
        j0 = base + ch * _DCH
        pltpu.sync_copy(pos_hbm.at[pl.ds(j0, _DCH)], idx_v)
        tok0 = jnp.where(j0 >= T, j0 - T, j0)
        cp = pltpu.async_copy(xf_hbm.at[pl.ds(tok0, _DCH)], rows_v, ld)
        cp.wait()
        scat[ch % 2] = pltpu.async_copy(rows_v, xs_hbm.at[idx_v], st)
    scat[0].wait()
    scat[1].wait()


def _gelu_exact(x):
    return 0.5 * x * (1.0 + lax.erf(x * 0.7071067811865476))


def _expert_changed(be_s, i):
    be = be_s[i]
    prev = be_s[jnp.maximum(i - 1, 0)]
    return jnp.logical_or(i == 0, be != prev)


def _ffn1_body(be_s, xs_ref, w1_ref, hdn_ref, w1c_ref):
    i = pl.program_id(0)
    active = be_s[i] < E

    @pl.when(jnp.logical_and(active, _expert_changed(be_s, i)))
    def _():
        w1c_ref[...] = w1_ref[...].astype(jnp.bfloat16)

    @pl.when(active)
    def _():
        xb = xs_ref[...].astype(jnp.bfloat16)
        hdn = lax.dot_general(xb, w1c_ref[...], (((1,), (1,)), ((), ())),
                              preferred_element_type=jnp.float32)
        hdn_ref[...] = _gelu_exact(hdn).astype(jnp.bfloat16)


def _ffn2_body(be_s, hdn_ref, w2_ref, ys_ref, w2c_ref):
    i = pl.program_id(0)
    active = be_s[i] < E

    @pl.when(jnp.logical_and(active, _expert_changed(be_s, i)))
    def _():
        w2c_ref[...] = w2_ref[...].astype(jnp.bfloat16)

    @pl.when(active)
    def _():
        ys_ref[...] = lax.dot_general(hdn_ref[...], w2c_ref[...],
                                      (((1,), (1,)), ((), ())),
                                      preferred_element_type=jnp.float32)


def _run_ffn(be, xs, w1r, w2r):
    hdn = pl.pallas_call(
        _ffn1_body,
        grid_spec=pltpu.PrefetchScalarGridSpec(
            num_scalar_prefetch=1,
            grid=(NBLK,),
            in_specs=[
                pl.BlockSpec((BLK, H), lambda i, be: (i, 0)),
                pl.BlockSpec((D_FF, H),
                             lambda i, be: (jnp.minimum(be[i], E - 1), 0)),
            ],
            out_specs=pl.BlockSpec((BLK, D_FF), lambda i, be: (i, 0)),
            scratch_shapes=[pltpu.VMEM((D_FF, H), jnp.bfloat16)],
        ),
        out_shape=jax.ShapeDtypeStruct((NPAD, D_FF), jnp.bfloat16),
        compiler_params=pltpu.CompilerParams(
            dimension_semantics=("arbitrary",)),
    )(be, xs, w1r)
    return pl.pallas_call(
        _ffn2_body,
        grid_spec=pltpu.PrefetchScalarGridSpec(
            num_scalar_prefetch=1,
            grid=(NBLK,),
            in_specs=[
                pl.BlockSpec((BLK, D_FF), lambda i, be: (i, 0)),
                pl.BlockSpec((H, D_FF),
                             lambda i, be: (jnp.minimum(be[i], E - 1), 0)),
            ],
            out_specs=pl.BlockSpec((BLK, H), lambda i, be: (i, 0)),
            scratch_shapes=[pltpu.VMEM((H, D_FF), jnp.bfloat16)],
        ),
        out_shape=jax.ShapeDtypeStruct((NPAD, H), jnp.float32),
        compiler_params=pltpu.CompilerParams(
            dimension_semantics=("arbitrary",)),
    )(be, hdn, w2r)


_CCH = 16
_TPW = T // _NW


@functools.lru_cache(maxsize=None)
def _make_combine():
    mesh = plsc.VectorSubcoreMesh(
        core_axis_name="c", subcore_axis_name="s",
        num_cores=_NC, num_subcores=_NS)
    per_set = [
        pltpu.VMEM((_CCH,), jnp.int32),
        pltpu.VMEM((_CCH,), jnp.int32),
        pltpu.VMEM((_CCH,), jnp.float32),
        pltpu.VMEM((_CCH,), jnp.float32),
        pltpu.VMEM((_CCH, H), jnp.float32),
        pltpu.VMEM((_CCH, H), jnp.float32),
        pltpu.SemaphoreType.DMA,
        pltpu.SemaphoreType.DMA,
    ]
    return pl.kernel(
        _combine_body,
        out_type=jax.ShapeDtypeStruct((T, H), jnp.float32),
        mesh=mesh,
        scratch_types=per_set + per_set,
    )


def _combine_body(pos_hbm, rw_hbm, ys_hbm, out_hbm, *scr):
    wid = lax.axis_index("s") * _NC + lax.axis_index("c")
    nch = _TPW // _CCH
    sets = (scr[:8], scr[8:])
    gath = [None, None, None, None]
    outw = [None, None]

    def issue(ch):
        i0_v, i1_v, w0_v, w1_v, a_v, b_v, sg, _so = sets[ch % 2]
        tb = wid * _TPW + ch * _CCH
        pltpu.sync_copy(pos_hbm.at[pl.ds(tb, _CCH)], i0_v)
        pltpu.sync_copy(pos_hbm.at[pl.ds(T + tb, _CCH)], i1_v)
        pltpu.sync_copy(rw_hbm.at[pl.ds(tb, _CCH)], w0_v)
        pltpu.sync_copy(rw_hbm.at[pl.ds(T + tb, _CCH)], w1_v)
        gath[2 * (ch % 2)] = pltpu.async_copy(ys_hbm.at[i0_v], a_v, sg)
        gath[2 * (ch % 2) + 1] = pltpu.async_copy(ys_hbm.at[i1_v], b_v, sg)

    def process(ch):
        _i0, _i1, w0_v, w1_v, a_v, b_v, _sg, so = sets[ch % 2]
        tb = wid * _TPW + ch * _CCH
        gath[2 * (ch % 2)].wait()
        gath[2 * (ch % 2) + 1].wait()
        wa = w0_v[...]
        wb = w1_v[...]
        for r in range(_CCH):
            ridx = jnp.full((16,), r, jnp.int32)
            sa = wa.at[ridx].get(mode="promise_in_bounds")
            sb = wb.at[ridx].get(mode="promise_in_bounds")

            def body(cc, _, r=r, sa=sa, sb=sb):
                col = cc * 64
                for u in range(4):
                    av = a_v[r, pl.ds(col + u * 16, 16)]
                    bv = b_v[r, pl.ds(col + u * 16, 16)]
                    a_v[r, pl.ds(col + u * 16, 16)] = sa * av + sb * bv
                return 0

            lax.fori_loop(0, H // 64, body, 0)
        outw[ch % 2] = pltpu.async_copy(a_v, out_hbm.at[pl.ds(tb, _CCH)], so)

    for ch in range(nch + 1):
        if ch < nch:
            if ch >= 2:
                outw[ch % 2].wait()
            issue(ch)
        if ch >= 1:
            process(ch - 1)
    outw[(nch - 2) % 2].wait()
    outw[(nch - 1) % 2].wait()


def kernel(x, gate_w, w1, w2):
    b, s, h = x.shape
    xf = x.reshape(T, H)
    pos, rw, be, aux = _run_router(xf, gate_w)
    pos1 = pos.reshape(N_ASSIGN)
    rw1 = rw.reshape(N_ASSIGN)
    be1 = be.reshape(NBLK)
    xs = _make_dispatch()(xf, pos1)
    ys = _run_ffn(be1, xs, w1.reshape(E * D_FF, H), w2.reshape(E * H, D_FF))
    out = _make_combine()(pos1, rw1, ys)
    return out.reshape(b, s, h), aux.reshape(())


# speedup vs baseline: 4.0204x; 1.1923x over previous
import functools

import jax
import jax.numpy as jnp
from jax import lax
from jax.experimental import pallas as pl
from jax.experimental.pallas import tpu as pltpu
from jax.experimental.pallas import tpu_sc as plsc

E = 8
TOP_K = 2
H = 1024
D_FF = 4096
T = 2048
N_ASSIGN = T * TOP_K
BLK = 256
NBLK = N_ASSIGN // BLK + E
NPAD = NBLK * BLK

_NC, _NS = 2, 16
_NW = _NC * _NS


def _router_body(x_ref, g_ref, pos_ref, rw_ref, be_ref, aux_ref):
    xf = x_ref[...]
    gw = g_ref[...]
    logits = lax.dot_general(xf, gw, (((1,), (1,)), ((), ())),
                             preferred_element_type=jnp.float32)
    m = jnp.max(logits, axis=1, keepdims=True)
    ex = jnp.exp(logits - m)
    probs = ex / jnp.sum(ex, axis=1, keepdims=True)

    iota_e = lax.broadcasted_iota(jnp.int32, (T, E), 1).astype(jnp.float32)
    m1 = jnp.max(probs, axis=1, keepdims=True)
    e1 = jnp.min(jnp.where(probs == m1, iota_e, 99.0), axis=1, keepdims=True)
    probs2 = jnp.where(iota_e == e1, -1.0, probs)
    m2 = jnp.max(probs2, axis=1, keepdims=True)
    e2 = jnp.min(jnp.where(probs2 == m2, iota_e, 99.0), axis=1, keepdims=True)
    ssum = m1 + m2
    rw_ref[...] = jnp.concatenate([m1 / ssum, m2 / ssum], axis=0)

    eall = jnp.concatenate([e1, e2], axis=0)
    iota_e2 = lax.broadcasted_iota(jnp.int32, (N_ASSIGN, E), 1).astype(jnp.float32)
    oh = (eall == iota_e2).astype(jnp.float32)

    c = oh
    sh = 1
    while sh < N_ASSIGN:
        c = c + jnp.concatenate(
            [jnp.zeros((sh, E), jnp.float32), c[:N_ASSIGN - sh]], axis=0)
        sh *= 2
    rank_incl = jnp.sum(c * oh, axis=1, keepdims=True)
    counts = c[N_ASSIGN - 1:N_ASSIGN, :]

    counts_i = counts.astype(jnp.int32)
    padded = (((counts_i + (BLK - 1)) >> 8) << 8).astype(jnp.float32)
    incl = padded
    for s2 in (1, 2, 4):
        incl = incl + jnp.concatenate(
            [jnp.zeros((1, s2), jnp.float32), incl[:, :E - s2]], axis=1)
    excl = incl - padded
    base = jnp.sum(oh * excl, axis=1, keepdims=True)
    pos_ref[...] = (base + rank_incl - 1.0).astype(jnp.int32)

    blk_start = lax.broadcasted_iota(jnp.int32, (NBLK, E), 0).astype(jnp.float32) * BLK
    becnt = jnp.sum((blk_start >= incl).astype(jnp.float32), axis=1,
                    keepdims=True)
    total = incl[0:1, E - 1:E]
    active = blk_start[:, 0:1] < total
    bevec = jnp.where(active, becnt, 15.0)
    prev = jnp.concatenate(
        [jnp.full((1, 1), -1.0, jnp.float32), bevec[:NBLK - 1]], axis=0)
    fog = jnp.where(jnp.logical_and(active, bevec != prev), 1.0, 0.0)
    grp = fog
    for s3 in (1, 2, 4, 8, 16):
        grp = grp + jnp.concatenate(
            [jnp.zeros((s3, 1), jnp.float32), grp[:NBLK - s3]], axis=0)
    grp = grp - 1.0
    par = grp - 2.0 * jnp.floor(grp * 0.5)
    cols = []
    run = jnp.full((1, 1), 15.0, jnp.float32)
    for e in range(E - 1, -1, -1):
        cols.append(run)
        run = jnp.where(padded[:, e:e + 1] > 0.0, float(e), run)
    nxt = jnp.concatenate(cols[::-1], axis=1)
    oh_blk = (becnt == lax.broadcasted_iota(jnp.int32, (NBLK, E), 1)
              .astype(jnp.float32)).astype(jnp.float32)
    pf = jnp.where(fog > 0.0,
                   jnp.sum(oh_blk * nxt, axis=1, keepdims=True), 15.0)
    be_ref[...] = jnp.concatenate([bevec, par, pf, fog],
                                  axis=1).astype(jnp.int32)

    pmean = jnp.sum(probs, axis=0, keepdims=True) * (1.0 / T)
    f_i = counts * (1.0 / T)
    aux_ref[...] = E * jnp.sum(f_i * pmean, axis=1, keepdims=True)


def _run_router(xf, gate_w):
    return pl.pallas_call(
        _router_body,
        out_shape=(
            jax.ShapeDtypeStruct((N_ASSIGN, 1), jnp.int32),
            jax.ShapeDtypeStruct((N_ASSIGN, 1), jnp.float32),
            jax.ShapeDtypeStruct((NBLK, 4), jnp.int32),
            jax.ShapeDtypeStruct((1, 1), jnp.float32),
        ),
    )(xf, gate_w)


_DCH = 32
_PER_W = N_ASSIGN // _NW

@functools.lru_cache(maxsize=None)
def _make_dispatch():
    mesh = plsc.VectorSubcoreMesh(
        core_axis_name="c", subcore_axis_name="s",
        num_cores=_NC, num_subcores=_NS)
    return pl.kernel(
        _dispatch_body,
        out_type=jax.ShapeDtypeStruct((NPAD, H), jnp.float32),
        mesh=mesh,
        scratch_types=[
            pltpu.VMEM((_DCH,), jnp.int32),
            pltpu.VMEM((_DCH,), jnp.int32),
            pltpu.VMEM((_DCH, H), jnp.float32),
            pltpu.VMEM((_DCH, H), jnp.float32),
            pltpu.SemaphoreType.DMA,
            pltpu.SemaphoreType.DMA,
            pltpu.SemaphoreType.DMA,
            pltpu.SemaphoreType.DMA,
        ],
    )


def _dispatch_body(xf_hbm, pos_hbm, xs_hbm,
                   idx0_v, idx1_v, rows0_v, rows1_v, ld0, ld1, st0, st1):
    wid = lax.axis_index("s") * _NC + lax.axis_index("c")
    base = wid * _PER_W
    nch = _PER_W // _DCH
    bufs = ((idx0_v, rows0_v, ld0, st0), (idx1_v, rows1_v, ld1, st1))
    scat = [None, None]
    for ch in range(nch):
        idx_v, rows_v, ld, st = bufs[ch % 2]
        if ch >= 2:
            scat[ch % 2].wait()
        j0 = base + ch * _DCH
        pltpu.sync_copy(pos_hbm.at[pl.ds(j0, _DCH)], idx_v)
        tok0 = jnp.where(j0 >= T, j0 - T, j0)
        cp = pltpu.async_copy(xf_hbm.at[pl.ds(tok0, _DCH)], rows_v, ld)
        cp.wait()
        scat[ch % 2] = pltpu.async_copy(rows_v, xs_hbm.at[idx_v], st)
    scat[0].wait()
    scat[1].wait()


def _gelu_exact(x):
    return 0.5 * x * (1.0 + lax.erf(x * 0.7071067811865476))


def _weight_stage(sc_s, w_hbm, wb0, wb1, wc_ref, s0, s1, rows):
    i = pl.program_id(0)
    be = sc_s[i, 0]
    par = sc_s[i, 1]
    pf = sc_s[i, 2]
    fog = sc_s[i, 3]

    @pl.when(i == 0)
    def _():
        cp = pltpu.make_async_copy(w_hbm.at[pl.ds(be * rows, rows)], wb0, s0)
        cp.start()
        cp.wait()

    @pl.when(jnp.logical_and(pf < E, par == 0))
    def _():
        pltpu.make_async_copy(w_hbm.at[pl.ds(pf * rows, rows)], wb1, s1).start()

    @pl.when(jnp.logical_and(pf < E, par == 1))
    def _():
        pltpu.make_async_copy(w_hbm.at[pl.ds(pf * rows, rows)], wb0, s0).start()

    started = jnp.logical_and(fog == 1, i > 0)

    @pl.when(jnp.logical_and(started, par == 0))
    def _():
        pltpu.make_async_copy(w_hbm.at[pl.ds(be * rows, rows)], wb0, s0).wait()

    @pl.when(jnp.logical_and(started, par == 1))
    def _():
        pltpu.make_async_copy(w_hbm.at[pl.ds(be * rows, rows)], wb1, s1).wait()

    @pl.when(jnp.logical_and(fog == 1, par == 0))
    def _():
        wc_ref[...] = wb0[...].astype(jnp.bfloat16)

    @pl.when(jnp.logical_and(fog == 1, par == 1))
    def _():
        wc_ref[...] = wb1[...].astype(jnp.bfloat16)


def _ffn1_body(sc_s, xs_ref, w1_hbm, hdn_ref, wb0, wb1, wc_ref, s0, s1):
    _weight_stage(sc_s, w1_hbm, wb0, wb1, wc_ref, s0, s1, D_FF)
    i = pl.program_id(0)

    @pl.when(sc_s[i, 0] < E)
    def _():
        xb = xs_ref[...].astype(jnp.bfloat16)
        hdn = lax.dot_general(xb, wc_ref[...], (((1,), (1,)), ((), ())),
                              preferred_element_type=jnp.float32)
        hdn_ref[...] = _gelu_exact(hdn).astype(jnp.bfloat16)


def _ffn2_body(sc_s, hdn_ref, w2_hbm, ys_ref, wb0, wb1, wc_ref, s0, s1):
    _weight_stage(sc_s, w2_hbm, wb0, wb1, wc_ref, s0, s1, H)
    i = pl.program_id(0)

    @pl.when(sc_s[i, 0] < E)
    def _():
        ys_ref[...] = lax.dot_general(hdn_ref[...], wc_ref[...],
                                      (((1,), (1,)), ((), ())),
                                      preferred_element_type=jnp.float32)


def _run_ffn(sc, xs, w1r, w2r):
    hdn = pl.pallas_call(
        _ffn1_body,
        grid_spec=pltpu.PrefetchScalarGridSpec(
            num_scalar_prefetch=1,
            grid=(NBLK,),
            in_specs=[
                pl.BlockSpec((BLK, H), lambda i, sc: (i, 0)),
                pl.BlockSpec(memory_space=pltpu.MemorySpace.HBM),
            ],
            out_specs=pl.BlockSpec((BLK, D_FF), lambda i, sc: (i, 0)),
            scratch_shapes=[
                pltpu.VMEM((D_FF, H), jnp.float32),
                pltpu.VMEM((D_FF, H), jnp.float32),
                pltpu.VMEM((D_FF, H), jnp.bfloat16),
                pltpu.SemaphoreType.DMA,
                pltpu.SemaphoreType.DMA,
            ],
        ),
        out_shape=jax.ShapeDtypeStruct((NPAD, D_FF), jnp.bfloat16),
        compiler_params=pltpu.CompilerParams(
            dimension_semantics=("arbitrary",)),
    )(sc, xs, w1r)
    return pl.pallas_call(
        _ffn2_body,
        grid_spec=pltpu.PrefetchScalarGridSpec(
            num_scalar_prefetch=1,
            grid=(NBLK,),
            in_specs=[
                pl.BlockSpec((BLK, D_FF), lambda i, sc: (i, 0)),
                pl.BlockSpec(memory_space=pltpu.MemorySpace.HBM),
            ],
            out_specs=pl.BlockSpec((BLK, H), lambda i, sc: (i, 0)),
            scratch_shapes=[
                pltpu.VMEM((H, D_FF), jnp.float32),
                pltpu.VMEM((H, D_FF), jnp.float32),
                pltpu.VMEM((H, D_FF), jnp.bfloat16),
                pltpu.SemaphoreType.DMA,
                pltpu.SemaphoreType.DMA,
            ],
        ),
        out_shape=jax.ShapeDtypeStruct((NPAD, H), jnp.float32),
        compiler_params=pltpu.CompilerParams(
            dimension_semantics=("arbitrary",)),
    )(sc, hdn, w2r)


_CCH = 16
_TPW = T // _NW


@functools.lru_cache(maxsize=None)
def _make_combine():
    mesh = plsc.VectorSubcoreMesh(
        core_axis_name="c", subcore_axis_name="s",
        num_cores=_NC, num_subcores=_NS)
    per_set = [
        pltpu.VMEM((_CCH,), jnp.int32),
        pltpu.VMEM((_CCH,), jnp.int32),
        pltpu.VMEM((_CCH,), jnp.float32),
        pltpu.VMEM((_CCH,), jnp.float32),
        pltpu.VMEM((_CCH, H), jnp.float32),
        pltpu.VMEM((_CCH, H), jnp.float32),
        pltpu.SemaphoreType.DMA,
        pltpu.SemaphoreType.DMA,
    ]
    return pl.kernel(
        _combine_body,
        out_type=jax.ShapeDtypeStruct((T, H), jnp.float32),
        mesh=mesh,
        scratch_types=per_set + per_set,
    )


def _combine_body(pos_hbm, rw_hbm, ys_hbm, out_hbm, *scr):
    wid = lax.axis_index("s") * _NC + lax.axis_index("c")
    nch = _TPW // _CCH
    sets = (scr[:8], scr[8:])
    gath = [None, None, None, None]
    outw = [None, None]

    def issue(ch):
        i0_v, i1_v, w0_v, w1_v, a_v, b_v, sg, _so = sets[ch % 2]
        tb = wid * _TPW + ch * _CCH
        pltpu.sync_copy(pos_hbm.at[pl.ds(tb, _CCH)], i0_v)
        pltpu.sync_copy(pos_hbm.at[pl.ds(T + tb, _CCH)], i1_v)
        pltpu.sync_copy(rw_hbm.at[pl.ds(tb, _CCH)], w0_v)
        pltpu.sync_copy(rw_hbm.at[pl.ds(T + tb, _CCH)], w1_v)
        gath[2 * (ch % 2)] = pltpu.async_copy(ys_hbm.at[i0_v], a_v, sg)
        gath[2 * (ch % 2) + 1] = pltpu.async_copy(ys_hbm.at[i1_v], b_v, sg)

    def process(ch):
        _i0, _i1, w0_v, w1_v, a_v, b_v, _sg, so = sets[ch % 2]
        tb = wid * _TPW + ch * _CCH
        gath[2 * (ch % 2)].wait()
        gath[2 * (ch % 2) + 1].wait()
        wa = w0_v[...]
        wb = w1_v[...]
        for r in range(_CCH):
            ridx = jnp.full((16,), r, jnp.int32)
            sa = wa.at[ridx].get(mode="promise_in_bounds")
            sb = wb.at[ridx].get(mode="promise_in_bounds")

            def body(cc, _, r=r, sa=sa, sb=sb):
                col = cc * 64
                for u in range(4):
                    av = a_v[r, pl.ds(col + u * 16, 16)]
                    bv = b_v[r, pl.ds(col + u * 16, 16)]
                    a_v[r, pl.ds(col + u * 16, 16)] = sa * av + sb * bv
                return 0

            lax.fori_loop(0, H // 64, body, 0)
        outw[ch % 2] = pltpu.async_copy(a_v, out_hbm.at[pl.ds(tb, _CCH)], so)

    for ch in range(nch + 1):
        if ch < nch:
            if ch >= 2:
                outw[ch % 2].wait()
            issue(ch)
        if ch >= 1:
            process(ch - 1)
    outw[(nch - 2) % 2].wait()
    outw[(nch - 1) % 2].wait()


def kernel(x, gate_w, w1, w2):
    b, s, h = x.shape
    xf = x.reshape(T, H)
    pos, rw, sc, aux = _run_router(xf, gate_w)
    pos1 = pos.reshape(N_ASSIGN)
    rw1 = rw.reshape(N_ASSIGN)
    xs = _make_dispatch()(xf, pos1)
    ys = _run_ffn(sc, xs, w1.reshape(E * D_FF, H), w2.reshape(E * H, D_FF))
    out = _make_combine()(pos1, rw1, ys)
    return out.reshape(b, s, h), aux.reshape(())


# speedup vs baseline: 4.2251x; 1.0509x over previous
import functools

import jax
import jax.numpy as jnp
from jax import lax
from jax.experimental import pallas as pl
from jax.experimental.pallas import tpu as pltpu
from jax.experimental.pallas import tpu_sc as plsc

E = 8
TOP_K = 2
H = 1024
D_FF = 4096
T = 2048
N_ASSIGN = T * TOP_K
BLK = 256
NBLK = N_ASSIGN // BLK + E
NPAD = NBLK * BLK

_NC, _NS = 2, 16
_NW = _NC * _NS


def _router_body(x_ref, g_ref, pos_ref, rw_ref, be_ref, aux_ref):
    xf = x_ref[...]
    gw = g_ref[...]
    logits = lax.dot_general(xf, gw, (((1,), (1,)), ((), ())),
                             preferred_element_type=jnp.float32)
    m = jnp.max(logits, axis=1, keepdims=True)
    ex = jnp.exp(logits - m)
    probs = ex / jnp.sum(ex, axis=1, keepdims=True)

    iota_e = lax.broadcasted_iota(jnp.int32, (T, E), 1).astype(jnp.float32)
    m1 = jnp.max(probs, axis=1, keepdims=True)
    e1 = jnp.min(jnp.where(probs == m1, iota_e, 99.0), axis=1, keepdims=True)
    probs2 = jnp.where(iota_e == e1, -1.0, probs)
    m2 = jnp.max(probs2, axis=1, keepdims=True)
    e2 = jnp.min(jnp.where(probs2 == m2, iota_e, 99.0), axis=1, keepdims=True)
    ssum = m1 + m2
    rw_ref[...] = jnp.concatenate([m1 / ssum, m2 / ssum], axis=0)

    eall = jnp.concatenate([e1, e2], axis=0)
    iota_e2 = lax.broadcasted_iota(jnp.int32, (N_ASSIGN, E), 1).astype(jnp.float32)
    oh = (eall == iota_e2).astype(jnp.float32)

    c = oh
    sh = 1
    while sh < N_ASSIGN:
        c = c + jnp.concatenate(
            [jnp.zeros((sh, E), jnp.float32), c[:N_ASSIGN - sh]], axis=0)
        sh *= 2
    rank_incl = jnp.sum(c * oh, axis=1, keepdims=True)
    counts = c[N_ASSIGN - 1:N_ASSIGN, :]

    counts_i = counts.astype(jnp.int32)
    padded = (((counts_i + (BLK - 1)) >> 8) << 8).astype(jnp.float32)
    incl = padded
    for s2 in (1, 2, 4):
        incl = incl + jnp.concatenate(
            [jnp.zeros((1, s2), jnp.float32), incl[:, :E - s2]], axis=1)
    excl = incl - padded
    base = jnp.sum(oh * excl, axis=1, keepdims=True)
    pos_ref[...] = (base + rank_incl - 1.0).astype(jnp.int32)

    blk_start = lax.broadcasted_iota(jnp.int32, (NBLK, E), 0).astype(jnp.float32) * BLK
    becnt = jnp.sum((blk_start >= incl).astype(jnp.float32), axis=1,
                    keepdims=True)
    total = incl[0:1, E - 1:E]
    active = blk_start[:, 0:1] < total
    bevec = jnp.where(active, becnt, 15.0)
    prev = jnp.concatenate(
        [jnp.full((1, 1), -1.0, jnp.float32), bevec[:NBLK - 1]], axis=0)
    fog = jnp.where(jnp.logical_and(active, bevec != prev), 1.0, 0.0)
    grp = fog
    for s3 in (1, 2, 4, 8, 16):
        grp = grp + jnp.concatenate(
            [jnp.zeros((s3, 1), jnp.float32), grp[:NBLK - s3]], axis=0)
    grp = grp - 1.0
    par = grp - 2.0 * jnp.floor(grp * 0.5)
    cols = []
    run = jnp.full((1, 1), 15.0, jnp.float32)
    for e in range(E - 1, -1, -1):
        cols.append(run)
        run = jnp.where(padded[:, e:e + 1] > 0.0, float(e), run)
    nxt = jnp.concatenate(cols[::-1], axis=1)
    oh_blk = (becnt == lax.broadcasted_iota(jnp.int32, (NBLK, E), 1)
              .astype(jnp.float32)).astype(jnp.float32)
    pf = jnp.where(fog > 0.0,
                   jnp.sum(oh_blk * nxt, axis=1, keepdims=True), 15.0)
    be_ref[...] = jnp.concatenate([bevec, par, pf, fog],
                                  axis=1).astype(jnp.int32)

    pmean = jnp.sum(probs, axis=0, keepdims=True) * (1.0 / T)
    f_i = counts * (1.0 / T)
    aux_ref[...] = E * jnp.sum(f_i * pmean, axis=1, keepdims=True)


def _run_router(xf, gate_w):
    return pl.pallas_call(
        _router_body,
        out_shape=(
            jax.ShapeDtypeStruct((N_ASSIGN, 1), jnp.int32),
            jax.ShapeDtypeStruct((N_ASSIGN, 1), jnp.float32),
            jax.ShapeDtypeStruct((NBLK, 4), jnp.int32),
            jax.ShapeDtypeStruct((1, 1), jnp.float32),
        ),
    )(xf, gate_w)


_DCH = 32
_PER_W = N_ASSIGN // _NW

@functools.lru_cache(maxsize=None)
def _make_dispatch():
    mesh = plsc.VectorSubcoreMesh(
        core_axis_name="c", subcore_axis_name="s",
        num_cores=_NC, num_subcores=_NS)
    return pl.kernel(
        _dispatch_body,
        out_type=jax.ShapeDtypeStruct((NPAD, H), jnp.float32),
        mesh=mesh,
        scratch_types=[
            pltpu.VMEM((_DCH,), jnp.int32),
            pltpu.VMEM((_DCH,), jnp.int32),
            pltpu.VMEM((_DCH, H), jnp.float32),
            pltpu.VMEM((_DCH, H), jnp.float32),
            pltpu.SemaphoreType.DMA,
            pltpu.SemaphoreType.DMA,
            pltpu.SemaphoreType.DMA,
            pltpu.SemaphoreType.DMA,
        ],
    )


def _dispatch_body(xf_hbm, pos_hbm, xs_hbm,
                   idx0_v, idx1_v, rows0_v, rows1_v, ld0, ld1, st0, st1):
    wid = lax.axis_index("s") * _NC + lax.axis_index("c")
    base = wid * _PER_W
    nch = _PER_W // _DCH
    bufs = ((idx0_v, rows0_v, ld0, st0), (idx1_v, rows1_v, ld1, st1))
    scat = [None, None]
    for ch in range(nch):
        idx_v, rows_v, ld, st = bufs[ch % 2]
        if ch >= 2:
            scat[ch % 2].wait()
        j0 = base + ch * _DCH
        pltpu.sync_copy(pos_hbm.at[pl.ds(j0, _DCH)], idx_v)
        tok0 = jnp.where(j0 >= T, j0 - T, j0)
        cp = pltpu.async_copy(xf_hbm.at[pl.ds(tok0, _DCH)], rows_v, ld)
        cp.wait()
        scat[ch % 2] = pltpu.async_copy(rows_v, xs_hbm.at[idx_v], st)
    scat[0].wait()
    scat[1].wait()


def _gelu_exact(x):
    return 0.5 * x * (1.0 + lax.erf(x * 0.7071067811865476))


def _weight_stage(sc_s, w_hbm, wb0, wb1, wc_ref, s0, s1, rows):
    i = pl.program_id(0)
    be = sc_s[i, 0]
    par = sc_s[i, 1]
    pf = sc_s[i, 2]
    fog = sc_s[i, 3]

    @pl.when(i == 0)
    def _():
        cp = pltpu.make_async_copy(w_hbm.at[pl.ds(be * rows, rows)], wb0, s0)
        cp.start()
        cp.wait()

    @pl.when(jnp.logical_and(pf < E, par == 0))
    def _():
        pltpu.make_async_copy(w_hbm.at[pl.ds(pf * rows, rows)], wb1, s1).start()

    @pl.when(jnp.logical_and(pf < E, par == 1))
    def _():
        pltpu.make_async_copy(w_hbm.at[pl.ds(pf * rows, rows)], wb0, s0).start()

    started = jnp.logical_and(fog == 1, i > 0)

    @pl.when(jnp.logical_and(started, par == 0))
    def _():
        pltpu.make_async_copy(w_hbm.at[pl.ds(be * rows, rows)], wb0, s0).wait()

    @pl.when(jnp.logical_and(started, par == 1))
    def _():
        pltpu.make_async_copy(w_hbm.at[pl.ds(be * rows, rows)], wb1, s1).wait()

    @pl.when(jnp.logical_and(fog == 1, par == 0))
    def _():
        wc_ref[...] = wb0[...].astype(jnp.bfloat16)

    @pl.when(jnp.logical_and(fog == 1, par == 1))
    def _():
        wc_ref[...] = wb1[...].astype(jnp.bfloat16)


def _ffn1_body(sc_s, xs_ref, w1_hbm, hdn_ref, wb0, wb1, wc_ref, s0, s1):
    _weight_stage(sc_s, w1_hbm, wb0, wb1, wc_ref, s0, s1, D_FF)
    i = pl.program_id(0)

    @pl.when(sc_s[i, 0] < E)
    def _():
        xb = xs_ref[...].astype(jnp.bfloat16)
        hdn = lax.dot_general(xb, wc_ref[...], (((1,), (1,)), ((), ())),
                              preferred_element_type=jnp.float32)
        hdn_ref[...] = _gelu_exact(hdn).astype(jnp.bfloat16)


def _ffn2_body(sc_s, hdn_ref, w2_hbm, ys_ref, wb0, wb1, wc_ref, s0, s1):
    _weight_stage(sc_s, w2_hbm, wb0, wb1, wc_ref, s0, s1, H)
    i = pl.program_id(0)

    @pl.when(sc_s[i, 0] < E)
    def _():
        ys_ref[...] = lax.dot_general(hdn_ref[...], wc_ref[...],
                                      (((1,), (1,)), ((), ())),
                                      preferred_element_type=jnp.float32)


def _run_ffn(sc, xs, w1r, w2r):
    hdn = pl.pallas_call(
        _ffn1_body,
        grid_spec=pltpu.PrefetchScalarGridSpec(
            num_scalar_prefetch=1,
            grid=(NBLK,),
            in_specs=[
                pl.BlockSpec((BLK, H), lambda i, sc: (i, 0)),
                pl.BlockSpec(memory_space=pltpu.MemorySpace.HBM),
            ],
            out_specs=pl.BlockSpec((BLK, D_FF), lambda i, sc: (i, 0)),
            scratch_shapes=[
                pltpu.VMEM((D_FF, H), jnp.float32),
                pltpu.VMEM((D_FF, H), jnp.float32),
                pltpu.VMEM((D_FF, H), jnp.bfloat16),
                pltpu.SemaphoreType.DMA,
                pltpu.SemaphoreType.DMA,
            ],
        ),
        out_shape=jax.ShapeDtypeStruct((NPAD, D_FF), jnp.bfloat16),
        compiler_params=pltpu.CompilerParams(
            dimension_semantics=("arbitrary",)),
    )(sc, xs, w1r)
    return pl.pallas_call(
        _ffn2_body,
        grid_spec=pltpu.PrefetchScalarGridSpec(
            num_scalar_prefetch=1,
            grid=(NBLK,),
            in_specs=[
                pl.BlockSpec((BLK, D_FF), lambda i, sc: (i, 0)),
                pl.BlockSpec(memory_space=pltpu.MemorySpace.HBM),
            ],
            out_specs=pl.BlockSpec((BLK, H), lambda i, sc: (i, 0)),
            scratch_shapes=[
                pltpu.VMEM((H, D_FF), jnp.float32),
                pltpu.VMEM((H, D_FF), jnp.float32),
                pltpu.VMEM((H, D_FF), jnp.bfloat16),
                pltpu.SemaphoreType.DMA,
                pltpu.SemaphoreType.DMA,
            ],
        ),
        out_shape=jax.ShapeDtypeStruct((NPAD, H), jnp.float32),
        compiler_params=pltpu.CompilerParams(
            dimension_semantics=("arbitrary",)),
    )(sc, hdn, w2r)


_CCH = 16
_TPW = T // _NW


@functools.lru_cache(maxsize=None)
def _make_combine():
    mesh = plsc.VectorSubcoreMesh(
        core_axis_name="c", subcore_axis_name="s",
        num_cores=_NC, num_subcores=_NS)
    per_set = [
        pltpu.VMEM((_CCH,), jnp.int32),
        pltpu.VMEM((_CCH,), jnp.int32),
        pltpu.VMEM((_CCH,), jnp.float32),
        pltpu.VMEM((_CCH,), jnp.float32),
        pltpu.VMEM((_CCH, H), jnp.float32),
        pltpu.VMEM((_CCH, H), jnp.float32),
        pltpu.SemaphoreType.DMA,
        pltpu.SemaphoreType.DMA,
    ]
    return pl.kernel(
        _combine_body,
        out_type=jax.ShapeDtypeStruct((T, H), jnp.float32),
        mesh=mesh,
        scratch_types=per_set + per_set,
    )


def _combine_body(pos_hbm, rw_hbm, ys_hbm, out_hbm, *scr):
    wid = lax.axis_index("s") * _NC + lax.axis_index("c")
    nch = _TPW // _CCH
    sets = (scr[:8], scr[8:])
    gath = [None, None, None, None]
    outw = [None, None]

    def issue(ch):
        i0_v, i1_v, w0_v, w1_v, a_v, b_v, sg, _so = sets[ch % 2]
        tb = wid * _TPW + ch * _CCH
        pltpu.sync_copy(pos_hbm.at[pl.ds(tb, _CCH)], i0_v)
        pltpu.sync_copy(pos_hbm.at[pl.ds(T + tb, _CCH)], i1_v)
        pltpu.sync_copy(rw_hbm.at[pl.ds(tb, _CCH)], w0_v)
        pltpu.sync_copy(rw_hbm.at[pl.ds(T + tb, _CCH)], w1_v)
        gath[2 * (ch % 2)] = pltpu.async_copy(ys_hbm.at[i0_v], a_v, sg)
        gath[2 * (ch % 2) + 1] = pltpu.async_copy(ys_hbm.at[i1_v], b_v, sg)

    def process(ch):
        _i0, _i1, w0_v, w1_v, a_v, b_v, _sg, so = sets[ch % 2]
        tb = wid * _TPW + ch * _CCH
        gath[2 * (ch % 2)].wait()
        gath[2 * (ch % 2) + 1].wait()
        wa = w0_v[...]
        wb = w1_v[...]
        for r in range(_CCH):
            ridx = jnp.full((16,), r, jnp.int32)
            sa = wa.at[ridx].get(mode="promise_in_bounds")
            sb = wb.at[ridx].get(mode="promise_in_bounds")

            @plsc.parallel_loop(0, H // 16, 1, unroll=8)
            def _(cc, r=r, sa=sa, sb=sb):
                av = a_v[r, pl.ds(cc * 16, 16)]
                bv = b_v[r, pl.ds(cc * 16, 16)]
                a_v[r, pl.ds(cc * 16, 16)] = sa * av + sb * bv
        outw[ch % 2] = pltpu.async_copy(a_v, out_hbm.at[pl.ds(tb, _CCH)], so)

    for ch in range(nch + 1):
        if ch < nch:
            if ch >= 2:
                outw[ch % 2].wait()
            issue(ch)
        if ch >= 1:
            process(ch - 1)
    outw[(nch - 2) % 2].wait()
    outw[(nch - 1) % 2].wait()


def kernel(x, gate_w, w1, w2):
    b, s, h = x.shape
    xf = x.reshape(T, H)
    pos, rw, sc, aux = _run_router(xf, gate_w)
    pos1 = pos.reshape(N_ASSIGN)
    rw1 = rw.reshape(N_ASSIGN)
    xs = _make_dispatch()(xf, pos1)
    ys = _run_ffn(sc, xs, w1.reshape(E * D_FF, H), w2.reshape(E * H, D_FF))
    out = _make_combine()(pos1, rw1, ys)
    return out.reshape(b, s, h), aux.reshape(())


# speedup vs baseline: 4.2312x; 1.0014x over previous
import functools

import jax
import jax.numpy as jnp
from jax import lax
from jax.experimental import pallas as pl
from jax.experimental.pallas import tpu as pltpu
from jax.experimental.pallas import tpu_sc as plsc

E = 8
TOP_K = 2
H = 1024
D_FF = 4096
T = 2048
N_ASSIGN = T * TOP_K
BLK = 256
NBLK = N_ASSIGN // BLK + E
NPAD = NBLK * BLK

_NC, _NS = 2, 16
_NW = _NC * _NS


def _router_body(x_ref, g_ref, pos_ref, rw_ref, be_ref, aux_ref):
    xf = x_ref[...]
    gw = g_ref[...]
    logits = lax.dot_general(xf, gw, (((1,), (1,)), ((), ())),
                             preferred_element_type=jnp.float32)
    m = jnp.max(logits, axis=1, keepdims=True)
    ex = jnp.exp(logits - m)
    probs = ex / jnp.sum(ex, axis=1, keepdims=True)

    iota_e = lax.broadcasted_iota(jnp.int32, (T, E), 1).astype(jnp.float32)
    m1 = jnp.max(probs, axis=1, keepdims=True)
    e1 = jnp.min(jnp.where(probs == m1, iota_e, 99.0), axis=1, keepdims=True)
    probs2 = jnp.where(iota_e == e1, -1.0, probs)
    m2 = jnp.max(probs2, axis=1, keepdims=True)
    e2 = jnp.min(jnp.where(probs2 == m2, iota_e, 99.0), axis=1, keepdims=True)
    ssum = m1 + m2
    rw_ref[...] = jnp.concatenate([m1 / ssum, m2 / ssum], axis=0)

    eall = jnp.concatenate([e1, e2], axis=0)
    iota_e2 = lax.broadcasted_iota(jnp.int32, (N_ASSIGN, E), 1).astype(jnp.float32)
    oh = (eall == iota_e2).astype(jnp.float32)

    c = oh
    sh = 1
    while sh < N_ASSIGN:
        c = c + jnp.concatenate(
            [jnp.zeros((sh, E), jnp.float32), c[:N_ASSIGN - sh]], axis=0)
        sh *= 2
    rank_incl = jnp.sum(c * oh, axis=1, keepdims=True)
    counts = c[N_ASSIGN - 1:N_ASSIGN, :]

    counts_i = counts.astype(jnp.int32)
    padded = (((counts_i + (BLK - 1)) >> 8) << 8).astype(jnp.float32)
    incl = padded
    for s2 in (1, 2, 4):
        incl = incl + jnp.concatenate(
            [jnp.zeros((1, s2), jnp.float32), incl[:, :E - s2]], axis=1)
    excl = incl - padded
    base = jnp.sum(oh * excl, axis=1, keepdims=True)
    pos_ref[...] = (base + rank_incl - 1.0).astype(jnp.int32)

    blk_start = lax.broadcasted_iota(jnp.int32, (NBLK, E), 0).astype(jnp.float32) * BLK
    becnt = jnp.sum((blk_start >= incl).astype(jnp.float32), axis=1,
                    keepdims=True)
    total = incl[0:1, E - 1:E]
    active = blk_start[:, 0:1] < total
    bevec = jnp.where(active, becnt, 15.0)
    prev = jnp.concatenate(
        [jnp.full((1, 1), -1.0, jnp.float32), bevec[:NBLK - 1]], axis=0)
    fog = jnp.where(jnp.logical_and(active, bevec != prev), 1.0, 0.0)
    grp = fog
    for s3 in (1, 2, 4, 8, 16):
        grp = grp + jnp.concatenate(
            [jnp.zeros((s3, 1), jnp.float32), grp[:NBLK - s3]], axis=0)
    grp = grp - 1.0
    par = grp - 2.0 * jnp.floor(grp * 0.5)
    cols = []
    run = jnp.full((1, 1), 15.0, jnp.float32)
    for e in range(E - 1, -1, -1):
        cols.append(run)
        run = jnp.where(padded[:, e:e + 1] > 0.0, float(e), run)
    nxt = jnp.concatenate(cols[::-1], axis=1)
    oh_blk = (becnt == lax.broadcasted_iota(jnp.int32, (NBLK, E), 1)
              .astype(jnp.float32)).astype(jnp.float32)
    pf = jnp.where(fog > 0.0,
                   jnp.sum(oh_blk * nxt, axis=1, keepdims=True), 15.0)
    be_ref[...] = jnp.concatenate([bevec, par, pf, fog],
                                  axis=1).astype(jnp.int32)

    pmean = jnp.sum(probs, axis=0, keepdims=True) * (1.0 / T)
    f_i = counts * (1.0 / T)
    aux_ref[...] = E * jnp.sum(f_i * pmean, axis=1, keepdims=True)


def _run_router(xf, gate_w):
    return pl.pallas_call(
        _router_body,
        out_shape=(
            jax.ShapeDtypeStruct((N_ASSIGN, 1), jnp.int32),
            jax.ShapeDtypeStruct((N_ASSIGN, 1), jnp.float32),
            jax.ShapeDtypeStruct((NBLK, 4), jnp.int32),
            jax.ShapeDtypeStruct((1, 1), jnp.float32),
        ),
    )(xf, gate_w)


_DCH = 32
_PER_W = N_ASSIGN // _NW

@functools.lru_cache(maxsize=None)
def _make_dispatch():
    mesh = plsc.VectorSubcoreMesh(
        core_axis_name="c", subcore_axis_name="s",
        num_cores=_NC, num_subcores=_NS)
    return pl.kernel(
        _dispatch_body,
        out_type=jax.ShapeDtypeStruct((NPAD, H), jnp.float32),
        mesh=mesh,
        scratch_types=[
            pltpu.VMEM((_PER_W // _DCH, _DCH), jnp.int32),
            pltpu.VMEM((_DCH, H), jnp.float32),
            pltpu.VMEM((_DCH, H), jnp.float32),
            pltpu.SemaphoreType.DMA,
            pltpu.SemaphoreType.DMA,
            pltpu.SemaphoreType.DMA,
            pltpu.SemaphoreType.DMA,
        ],
    )


def _dispatch_body(xf_hbm, pos_hbm, xs_hbm,
                   idx_v, rows0_v, rows1_v, ld0, ld1, st0, st1):
    wid = lax.axis_index("s") * _NC + lax.axis_index("c")
    base = wid * _PER_W
    nch = _PER_W // _DCH
    rows = (rows0_v, rows1_v)
    lds = (ld0, ld1)
    sts = (st0, st1)
    pltpu.sync_copy(pos_hbm.at[wid], idx_v)

    def tok0(ch):
        j0 = base + ch * _DCH
        return jnp.where(j0 >= T, j0 - T, j0)

    loads = [None, None]
    scat = [None, None]
    loads[0] = pltpu.async_copy(xf_hbm.at[pl.ds(tok0(0), _DCH)], rows0_v, ld0)
    for ch in range(nch):
        par = ch % 2
        loads[par].wait()
        if ch + 1 < nch:
            opar = 1 - par
            if ch >= 1:
                scat[opar].wait()
            loads[opar] = pltpu.async_copy(
                xf_hbm.at[pl.ds(tok0(ch + 1), _DCH)], rows[opar], lds[opar])
        scat[par] = pltpu.async_copy(rows[par], xs_hbm.at[idx_v.at[ch]],
                                     sts[par])
    scat[0].wait()
    scat[1].wait()


def _gelu_exact(x):
    return 0.5 * x * (1.0 + lax.erf(x * 0.7071067811865476))


def _weight_stage(sc_s, w_hbm, wb0, wb1, wc_ref, s0, s1, rows):
    i = pl.program_id(0)
    be = sc_s[i, 0]
    par = sc_s[i, 1]
    pf = sc_s[i, 2]
    fog = sc_s[i, 3]

    @pl.when(i == 0)
    def _():
        cp = pltpu.make_async_copy(w_hbm.at[pl.ds(be * rows, rows)], wb0, s0)
        cp.start()
        cp.wait()

    @pl.when(jnp.logical_and(pf < E, par == 0))
    def _():
        pltpu.make_async_copy(w_hbm.at[pl.ds(pf * rows, rows)], wb1, s1).start()

    @pl.when(jnp.logical_and(pf < E, par == 1))
    def _():
        pltpu.make_async_copy(w_hbm.at[pl.ds(pf * rows, rows)], wb0, s0).start()

    started = jnp.logical_and(fog == 1, i > 0)

    @pl.when(jnp.logical_and(started, par == 0))
    def _():
        pltpu.make_async_copy(w_hbm.at[pl.ds(be * rows, rows)], wb0, s0).wait()

    @pl.when(jnp.logical_and(started, par == 1))
    def _():
        pltpu.make_async_copy(w_hbm.at[pl.ds(be * rows, rows)], wb1, s1).wait()

    @pl.when(jnp.logical_and(fog == 1, par == 0))
    def _():
        wc_ref[...] = wb0[...].astype(jnp.bfloat16)

    @pl.when(jnp.logical_and(fog == 1, par == 1))
    def _():
        wc_ref[...] = wb1[...].astype(jnp.bfloat16)


def _ffn1_body(sc_s, xs_ref, w1_hbm, hdn_ref, wb0, wb1, wc_ref, s0, s1):
    _weight_stage(sc_s, w1_hbm, wb0, wb1, wc_ref, s0, s1, D_FF)
    i = pl.program_id(0)

    @pl.when(sc_s[i, 0] < E)
    def _():
        xb = xs_ref[...].astype(jnp.bfloat16)
        hdn = lax.dot_general(xb, wc_ref[...], (((1,), (1,)), ((), ())),
                              preferred_element_type=jnp.float32)
        hdn_ref[...] = _gelu_exact(hdn).astype(jnp.bfloat16)


def _ffn2_body(sc_s, hdn_ref, w2_hbm, ys_ref, wb0, wb1, wc_ref, s0, s1):
    _weight_stage(sc_s, w2_hbm, wb0, wb1, wc_ref, s0, s1, H)
    i = pl.program_id(0)

    @pl.when(sc_s[i, 0] < E)
    def _():
        ys_ref[...] = lax.dot_general(hdn_ref[...], wc_ref[...],
                                      (((1,), (1,)), ((), ())),
                                      preferred_element_type=jnp.float32)


def _run_ffn(sc, xs, w1r, w2r):
    hdn = pl.pallas_call(
        _ffn1_body,
        grid_spec=pltpu.PrefetchScalarGridSpec(
            num_scalar_prefetch=1,
            grid=(NBLK,),
            in_specs=[
                pl.BlockSpec((BLK, H), lambda i, sc: (i, 0)),
                pl.BlockSpec(memory_space=pltpu.MemorySpace.HBM),
            ],
            out_specs=pl.BlockSpec((BLK, D_FF), lambda i, sc: (i, 0)),
            scratch_shapes=[
                pltpu.VMEM((D_FF, H), jnp.float32),
                pltpu.VMEM((D_FF, H), jnp.float32),
                pltpu.VMEM((D_FF, H), jnp.bfloat16),
                pltpu.SemaphoreType.DMA,
                pltpu.SemaphoreType.DMA,
            ],
        ),
        out_shape=jax.ShapeDtypeStruct((NPAD, D_FF), jnp.bfloat16),
        compiler_params=pltpu.CompilerParams(
            dimension_semantics=("arbitrary",)),
    )(sc, xs, w1r)
    return pl.pallas_call(
        _ffn2_body,
        grid_spec=pltpu.PrefetchScalarGridSpec(
            num_scalar_prefetch=1,
            grid=(NBLK,),
            in_specs=[
                pl.BlockSpec((BLK, D_FF), lambda i, sc: (i, 0)),
                pl.BlockSpec(memory_space=pltpu.MemorySpace.HBM),
            ],
            out_specs=pl.BlockSpec((BLK, H), lambda i, sc: (i, 0)),
            scratch_shapes=[
                pltpu.VMEM((H, D_FF), jnp.float32),
                pltpu.VMEM((H, D_FF), jnp.float32),
                pltpu.VMEM((H, D_FF), jnp.bfloat16),
                pltpu.SemaphoreType.DMA,
                pltpu.SemaphoreType.DMA,
            ],
        ),
        out_shape=jax.ShapeDtypeStruct((NPAD, H), jnp.float32),
        compiler_params=pltpu.CompilerParams(
            dimension_semantics=("arbitrary",)),
    )(sc, hdn, w2r)


_CCH = 16
_TPW = T // _NW


@functools.lru_cache(maxsize=None)
def _make_combine():
    mesh = plsc.VectorSubcoreMesh(
        core_axis_name="c", subcore_axis_name="s",
        num_cores=_NC, num_subcores=_NS)
    per_set = [
        pltpu.VMEM((_CCH, H), jnp.float32),
        pltpu.VMEM((_CCH, H), jnp.float32),
        pltpu.SemaphoreType.DMA,
        pltpu.SemaphoreType.DMA,
    ]
    return pl.kernel(
        _combine_body,
        out_type=jax.ShapeDtypeStruct((T, H), jnp.float32),
        mesh=mesh,
        scratch_types=[
            pltpu.VMEM((_TPW,), jnp.int32),
            pltpu.VMEM((_TPW,), jnp.int32),
            pltpu.VMEM((_TPW,), jnp.float32),
            pltpu.VMEM((_TPW,), jnp.float32),
        ] + per_set + per_set,
    )


def _combine_body(pos_hbm, rw_hbm, ys_hbm, out_hbm,
                  i0a, i1a, w0a, w1a, *scr):
    wid = lax.axis_index("s") * _NC + lax.axis_index("c")
    nch = _TPW // _CCH
    sets = (scr[:4], scr[4:])
    gath = [None, None, None, None]
    outw = [None, None]
    tb0 = wid * _TPW
    pltpu.sync_copy(pos_hbm.at[pl.ds(tb0, _TPW)], i0a)
    pltpu.sync_copy(pos_hbm.at[pl.ds(T + tb0, _TPW)], i1a)
    pltpu.sync_copy(rw_hbm.at[pl.ds(tb0, _TPW)], w0a)
    pltpu.sync_copy(rw_hbm.at[pl.ds(T + tb0, _TPW)], w1a)

    def issue(ch):
        a_v, b_v, sg, _so = sets[ch % 2]
        gath[2 * (ch % 2)] = pltpu.async_copy(
            ys_hbm.at[i0a.at[pl.ds(ch * _CCH, _CCH)]], a_v, sg)
        gath[2 * (ch % 2) + 1] = pltpu.async_copy(
            ys_hbm.at[i1a.at[pl.ds(ch * _CCH, _CCH)]], b_v, sg)

    def process(ch):
        a_v, b_v, _sg, so = sets[ch % 2]
        tb = tb0 + ch * _CCH
        gath[2 * (ch % 2)].wait()
        gath[2 * (ch % 2) + 1].wait()
        wa = w0a[pl.ds(ch * _CCH, _CCH)]
        wb = w1a[pl.ds(ch * _CCH, _CCH)]
        for r in range(_CCH):
            ridx = jnp.full((16,), r, jnp.int32)
            sa = wa.at[ridx].get(mode="promise_in_bounds")
            sb = wb.at[ridx].get(mode="promise_in_bounds")

            @plsc.parallel_loop(0, H // 16, 1, unroll=8)
            def _(cc, r=r, sa=sa, sb=sb):
                av = a_v[r, pl.ds(cc * 16, 16)]
                bv = b_v[r, pl.ds(cc * 16, 16)]
                a_v[r, pl.ds(cc * 16, 16)] = sa * av + sb * bv
        outw[ch % 2] = pltpu.async_copy(a_v, out_hbm.at[pl.ds(tb, _CCH)], so)

    for ch in range(nch + 1):
        if ch < nch:
            if ch >= 2:
                outw[ch % 2].wait()
            issue(ch)
        if ch >= 1:
            process(ch - 1)
    outw[(nch - 2) % 2].wait()
    outw[(nch - 1) % 2].wait()


def kernel(x, gate_w, w1, w2):
    b, s, h = x.shape
    xf = x.reshape(T, H)
    pos, rw, sc, aux = _run_router(xf, gate_w)
    pos1 = pos.reshape(N_ASSIGN)
    rw1 = rw.reshape(N_ASSIGN)
    xs = _make_dispatch()(xf, pos1.reshape(_NW, _PER_W // _DCH, _DCH))
    ys = _run_ffn(sc, xs, w1.reshape(E * D_FF, H), w2.reshape(E * H, D_FF))
    out = _make_combine()(pos1, rw1, ys)
    return out.reshape(b, s, h), aux.reshape(())
